# baseline (speedup 1.0000x reference)
import functools

import jax
import jax.numpy as jnp
from jax import lax
from jax.experimental import pallas as pl
from jax.experimental.pallas import tpu as pltpu
from jax.experimental.pallas import tpu_sc as plsc

_D = 768
_LANES = 16
_NC = 2
_NS = 16
_NW = _NC * _NS
_C = 16
_NBUF = 3


def _emb_body(tok_hbm, posflat_hbm, tidx_hbm, pidx_hbm, out_hbm,
              posvm, tidx_v, pidx_v,
              t0, t1, t2,
              g0, g1, g2,
              s0, s1, s2, *, per_w):
    wid = lax.axis_index("s") * _NC + lax.axis_index("c")
    base = wid * per_w
    nch = per_w // _C
    tbuf = (t0, t1, t2)
    gsem = (g0, g1, g2)
    ssem = (s0, s1, s2)
    lane = lax.iota(jnp.int32, _LANES)
    zeros = jnp.zeros((_LANES,), jnp.int32)

    pltpu.sync_copy(posflat_hbm, posvm)
    pltpu.sync_copy(tidx_hbm.at[pl.ds(base, per_w)], tidx_v)
    pltpu.sync_copy(pidx_hbm.at[pl.ds(base, per_w)], pidx_v)

    _NSPLIT = 2
    _CS = _C // _NSPLIT

    def issue_gather(ci, s):
        for k in range(_NSPLIT):
            pltpu.async_copy(
                tok_hbm.at[tidx_v.at[pl.ds(ci * _C + k * _CS, _CS)]],
                tbuf[s].at[pl.ds(k * _CS, _CS)], gsem[s])

    def wait_gather(ci, s):
        for k in range(_NSPLIT):
            pltpu.make_async_copy(
                tok_hbm.at[tidx_v.at[pl.ds(ci * _C + k * _CS, _CS)]],
                tbuf[s].at[pl.ds(k * _CS, _CS)], gsem[s]).wait()

    def issue_scatter(ci, s):
        pltpu.async_copy(tbuf[s], out_hbm.at[pl.ds(base + ci * _C, _C)],
                         ssem[s])

    def wait_scatter(ci, s):
        pltpu.make_async_copy(tbuf[s],
                              out_hbm.at[pl.ds(base + ci * _C, _C)],
                              ssem[s]).wait()

    def add_chunk(ci, s):
        t = tbuf[s]

        @plsc.parallel_loop(0, _C, step=1, unroll=2)
        def _(i):
            pbro = plsc.load_gather(pidx_v, [zeros + (ci * _C + i)])
            pb = pbro * _D + lane
            for j in range(_D // _LANES):
                pv = plsc.load_gather(posvm, [pb + (j * _LANES)])
                plsc.addupdate(t.at[i, pl.ds(j * _LANES, _LANES)], pv)

    issue_gather(0, 0)
    issue_gather(1, 1)
    issue_gather(2, 2)
    wait_gather(0, 0)
    add_chunk(0, 0)
    issue_scatter(0, 0)
    wait_scatter(0, 0)
    issue_gather(3, 0)
    wait_gather(1, 1)
    add_chunk(1, 1)
    issue_scatter(1, 1)

    def outer(g, _):
        for sp in range(_NBUF):
            ci = 2 + g * _NBUF + sp
            s = (2 + sp) % _NBUF
            s2 = (sp + 1) % _NBUF
            wait_scatter(ci - 1, s2)
            issue_gather(ci + 2, s2)
            wait_gather(ci, s)
            add_chunk(ci, s)
            issue_scatter(ci, s)
        return ()

    lax.fori_loop(0, (nch - 4) // _NBUF, outer, (), unroll=False)

    for ci in (nch - 2, nch - 1):
        s = ci % _NBUF
        wait_scatter(ci - 1, (ci + 2) % _NBUF)
        wait_gather(ci, s)
        add_chunk(ci, s)
        issue_scatter(ci, s)
    wait_scatter(nch - 1, (nch - 1) % _NBUF)


def kernel(token_table, pos_table, tokens, positions):
    b, l = tokens.shape
    bt = b * l
    per_w = bt // _NW
    assert per_w % _C == 0 and (per_w // _C - 4) % _NBUF == 0

    tidx = tokens.reshape(bt).astype(jnp.int32)
    pidx = positions.reshape(bt).astype(jnp.int32)
    posflat = pos_table.reshape(-1)

    mesh = plsc.VectorSubcoreMesh(core_axis_name="c", subcore_axis_name="s")
    body = functools.partial(_emb_body, per_w=per_w)
    run = pl.kernel(
        body,
        mesh=mesh,
        compiler_params=pltpu.CompilerParams(needs_layout_passes=False),
        out_type=jax.ShapeDtypeStruct((bt, _D), jnp.float32),
        scratch_types=[
            pltpu.VMEM((pos_table.size,), jnp.float32),
            pltpu.VMEM((per_w,), jnp.int32),
            pltpu.VMEM((per_w,), jnp.int32),
        ] + [pltpu.VMEM((_C, _D), jnp.float32)] * _NBUF
          + [pltpu.SemaphoreType.DMA] * (2 * _NBUF),
    )
    out = run(token_table, posflat, tidx, pidx)
    return out.reshape(b, l, _D)


# speedup vs baseline: 1.0574x; 1.0574x over previous
import functools

import jax
import jax.numpy as jnp
from jax import lax
from jax.experimental import pallas as pl
from jax.experimental.pallas import tpu as pltpu
from jax.experimental.pallas import tpu_sc as plsc

_D = 768
_LANES = 16
_NC = 2
_NS = 16
_NW = _NC * _NS
_C = 16
_NBUF = 3


def _emb_body(tok_hbm, pos_hbm, tidx_hbm, pidx_hbm, out_hbm,
              tidx_v, pidx_v,
              t0, t1, t2, q0, q1, q2,
              g0, g1, g2,
              p0, p1, p2,
              s0, s1, s2, *, per_w):
    wid = lax.axis_index("s") * _NC + lax.axis_index("c")
    base = wid * per_w
    nch = per_w // _C
    tbuf = (t0, t1, t2)
    pbuf = (q0, q1, q2)
    gsem = (g0, g1, g2)
    psem = (p0, p1, p2)
    ssem = (s0, s1, s2)

    pltpu.sync_copy(tidx_hbm.at[pl.ds(base, per_w)], tidx_v)
    pltpu.sync_copy(pidx_hbm.at[pl.ds(base, per_w)], pidx_v)

    def issue_tok(ci, s):
        pltpu.async_copy(tok_hbm.at[tidx_v.at[pl.ds(ci * _C, _C)]],
                         tbuf[s], gsem[s])

    def wait_tok(ci, s):
        pltpu.make_async_copy(tok_hbm.at[tidx_v.at[pl.ds(ci * _C, _C)]],
                              tbuf[s], gsem[s]).wait()

    def issue_pos(ci, s):
        pltpu.async_copy(pos_hbm.at[pidx_v.at[pl.ds(ci * _C, _C)]],
                         pbuf[s], psem[s])

    def wait_pos(ci, s):
        pltpu.make_async_copy(pos_hbm.at[pidx_v.at[pl.ds(ci * _C, _C)]],
                              pbuf[s], psem[s]).wait()

    def add_chunk(s):
        t = tbuf[s]
        q = pbuf[s]

        @plsc.parallel_loop(0, _C, step=1, unroll=4)
        def _(i):
            for j in range(_D // _LANES):
                sl = pl.ds(j * _LANES, _LANES)
                plsc.addupdate(t.at[i, sl], q[i, sl])

    def issue_scatter(ci, s):
        pltpu.async_copy(tbuf[s], out_hbm.at[pl.ds(base + ci * _C, _C)],
                         ssem[s])

    def wait_scatter(ci, s):
        pltpu.make_async_copy(tbuf[s],
                              out_hbm.at[pl.ds(base + ci * _C, _C)],
                              ssem[s]).wait()

    def step(ci, s):
        if ci >= 1:
            wait_scatter(ci - 1, (ci + 2) % _NBUF)
        if ci + 2 < nch:
            issue_tok(ci + 2, (ci + 2) % _NBUF)
            issue_pos(ci + 2, (ci + 2) % _NBUF)
        wait_tok(ci, s)
        wait_pos(ci, s)
        add_chunk(s)
        issue_scatter(ci, s)

    issue_tok(0, 0)
    issue_pos(0, 0)
    issue_tok(1, 1)
    issue_pos(1, 1)
    step(0, 0)
    step(1, 1)

    def outer(g, _):
        for sp in range(_NBUF):
            ci = 2 + g * _NBUF + sp
            s = (2 + sp) % _NBUF
            wait_scatter(ci - 1, (s + 2) % _NBUF)
            issue_tok(ci + 2, (s + 2) % _NBUF)
            issue_pos(ci + 2, (s + 2) % _NBUF)
            wait_tok(ci, s)
            wait_pos(ci, s)
            add_chunk(s)
            issue_scatter(ci, s)
        return ()

    lax.fori_loop(0, (nch - 4) // _NBUF, outer, (), unroll=False)

    step(nch - 2, (nch - 2) % _NBUF)
    step(nch - 1, (nch - 1) % _NBUF)
    wait_scatter(nch - 1, (nch - 1) % _NBUF)


def kernel(token_table, pos_table, tokens, positions):
    b, l = tokens.shape
    bt = b * l
    per_w = bt // _NW
    assert per_w % _C == 0 and (per_w // _C - 4) % _NBUF == 0

    tidx = tokens.reshape(bt).astype(jnp.int32)
    pidx = positions.reshape(bt).astype(jnp.int32)

    mesh = plsc.VectorSubcoreMesh(core_axis_name="c", subcore_axis_name="s")
    body = functools.partial(_emb_body, per_w=per_w)
    run = pl.kernel(
        body,
        mesh=mesh,
        compiler_params=pltpu.CompilerParams(needs_layout_passes=False),
        out_type=jax.ShapeDtypeStruct((bt, _D), jnp.float32),
        scratch_types=[
            pltpu.VMEM((per_w,), jnp.int32),
            pltpu.VMEM((per_w,), jnp.int32),
        ] + [pltpu.VMEM((_C, _D), jnp.float32)] * (2 * _NBUF)
          + [pltpu.SemaphoreType.DMA] * (3 * _NBUF),
    )
    out = run(token_table, pos_table, tidx, pidx)
    return out.reshape(b, l, _D)


# speedup vs baseline: 1.6934x; 1.6015x over previous
import functools

import jax
import jax.numpy as jnp
from jax import lax
from jax.experimental import pallas as pl
from jax.experimental.pallas import tpu as pltpu
from jax.experimental.pallas import tpu_sc as plsc

_D = 768
_LANES = 16
_NC = 2
_NS = 16
_NW = _NC * _NS
_C = 16
_NBUF = 3
_BB = 16


def _gather_body(tok_hbm, tidx_hbm, out_hbm, tidx_v,
                 t0, t1, t2, g0, g1, g2, s0, s1, s2, *, per_w, ll, lp):
    wid = lax.axis_index("s") * _NC + lax.axis_index("c")
    base = wid * per_w
    nch = per_w // _C
    lane = lax.iota(jnp.int32, _LANES)
    tbuf = (t0, t1, t2)
    gsem = (g0, g1, g2)
    ssem = (s0, s1, s2)

    pltpu.sync_copy(tidx_hbm.at[pl.ds(base, per_w)], tidx_v)

    def issue_tok(ci, s):
        pltpu.async_copy(tok_hbm.at[tidx_v.at[pl.ds(ci * _C, _C)]],
                         tbuf[s], gsem[s])

    def wait_tok(ci, s):
        pltpu.make_async_copy(tok_hbm.at[tidx_v.at[pl.ds(ci * _C, _C)]],
                              tbuf[s], gsem[s]).wait()

    def out_rows(ci):
        r = base + ci * _C + lane
        bv = r // ll
        return bv * lp + (r - bv * ll)

    def issue_scatter(ci, s):
        pltpu.async_copy(tbuf[s], out_hbm.at[out_rows(ci)], ssem[s])

    def wait_scatter(ci, s):
        pltpu.make_async_copy(tbuf[s], out_hbm.at[out_rows(ci)],
                              ssem[s]).wait()

    def step(ci, s):
        if ci >= 1:
            wait_scatter(ci - 1, (ci + 2) % _NBUF)
        if ci + 2 < nch:
            issue_tok(ci + 2, (ci + 2) % _NBUF)
        wait_tok(ci, s)
        issue_scatter(ci, s)

    issue_tok(0, 0)
    issue_tok(1, 1)
    step(0, 0)
    step(1, 1)

    def outer(g, _):
        for sp in range(_NBUF):
            ci = 2 + g * _NBUF + sp
            s = (2 + sp) % _NBUF
            wait_scatter(ci - 1, (s + 2) % _NBUF)
            issue_tok(ci + 2, (s + 2) % _NBUF)
            wait_tok(ci, s)
            issue_scatter(ci, s)
        return ()

    lax.fori_loop(0, (nch - 4) // _NBUF, outer, (), unroll=False)

    step(nch - 2, (nch - 2) % _NBUF)
    step(nch - 1, (nch - 1) % _NBUF)
    wait_scatter(nch - 1, (nch - 1) % _NBUF)


def _add_body(pos_ref, ptab_ref, tok_ref, out_ref, *, lp):
    ll = ptab_ref.shape[0]
    bb = out_ref.shape[0]
    posf = pos_ref[...]
    iota = lax.broadcasted_iota(jnp.int32, (1, ll), 1)
    onehot = (posf == iota).astype(jnp.float32)
    pe = lax.dot_general(onehot, ptab_ref[...], (((1,), (0,)), ((), ())),
                         preferred_element_type=jnp.float32)
    sm = (tok_ref[...] + pe).reshape(bb, lp, _D)
    out_ref[...] = sm[:, :ll, :]


def kernel(token_table, pos_table, tokens, positions):
    b, l = tokens.shape
    lp = (l + 7) // 8 * 8
    bt = b * l
    per_w = bt // _NW
    assert per_w % _C == 0 and (per_w // _C - 4) % _NBUF == 0

    tidx = tokens.reshape(bt).astype(jnp.int32)

    mesh = plsc.VectorSubcoreMesh(core_axis_name="c", subcore_axis_name="s")
    gather = pl.kernel(
        functools.partial(_gather_body, per_w=per_w, ll=l, lp=lp),
        mesh=mesh,
        compiler_params=pltpu.CompilerParams(needs_layout_passes=False),
        out_type=jax.ShapeDtypeStruct((b * lp, _D), jnp.float32),
        scratch_types=[
            pltpu.VMEM((per_w,), jnp.int32),
        ] + [pltpu.VMEM((_C, _D), jnp.float32)] * _NBUF
          + [pltpu.SemaphoreType.DMA] * (2 * _NBUF),
    )
    tok_rows = gather(token_table, tidx)

    pos_pad = jnp.pad(positions.astype(jnp.int32),
                      ((0, 0), (0, lp - l))).reshape(b * lp, 1)

    add = pl.pallas_call(
        functools.partial(_add_body, lp=lp),
        grid=(b // _BB,),
        in_specs=[
            pl.BlockSpec((_BB * lp, 1), lambda i: (i, 0)),
            pl.BlockSpec((l, _D), lambda i: (0, 0)),
            pl.BlockSpec((_BB * lp, _D), lambda i: (i, 0)),
        ],
        out_specs=pl.BlockSpec((_BB, l, _D), lambda i: (i, 0, 0)),
        out_shape=jax.ShapeDtypeStruct((b, l, _D), jnp.float32),
    )
    return add(pos_pad, pos_table, tok_rows)
